# Initial kernel scaffold; baseline (speedup 1.0000x reference)
import jax
import jax.numpy as jnp
from jax.experimental import pallas as pl


def kernel(x, edge_index, W1, b1, W2, b2, W3, b3):
    raise NotImplementedError("write your pallas kernel here")


# speedup vs baseline: 17.8737x; 17.8737x over previous
import dataclasses
import functools

import jax
import jax.numpy as jnp
from jax import lax
from jax.experimental import pallas as pl
from jax.experimental.pallas import tpu as pltpu
from jax.experimental.pallas import tpu_sc as plsc

N = 10000
E = 320000
D = 128

NC = 2
NS = 16
W = 128
NTILE = NC * NS
CH = -(-E // (NTILE * W))
EPAD = NTILE * W * CH
PAD_ROWS = 240
NTOT = N + PAD_ROWS
RPS = NTOT // NS

_mesh = plsc.VectorSubcoreMesh(core_axis_name="c", subcore_axis_name="s")

_cp = pltpu.CompilerParams()
if "needs_layout_passes" in pltpu.CompilerParams.__dataclass_fields__:
    _cp = dataclasses.replace(_cp, needs_layout_passes=False)


@jax.jit
def _sc_degree_vmem(dst3):
    @functools.partial(
        pl.kernel,
        out_type=jax.ShapeDtypeStruct((NTILE, NTOT), jnp.float32),
        mesh=_mesh,
        compiler_params=_cp,
        scratch_types=[
            pltpu.VMEM((CH, W), jnp.int32),
            pltpu.VMEM((NTOT,), jnp.float32),
        ],
    )
    def k(dst_hbm, out_hbm, dst_v, deg_v):
        c = lax.axis_index("c")
        s = lax.axis_index("s")
        tid = c * NS + s
        pltpu.sync_copy(dst_hbm.at[tid], dst_v)

        @pl.loop(0, NTOT // 16)
        def _(i):
            deg_v[pl.ds(i * 16, 16)] = jnp.zeros((16,), jnp.float32)

        ones = jnp.ones((16,), jnp.float32)

        @pl.loop(0, CH)
        def _(j):
            @pl.loop(0, W // 16)
            def _(kk):
                dv = dst_v[j, pl.ds(kk * 16, 16)]
                plsc.addupdate_scatter(deg_v, [dv], ones)

        pltpu.sync_copy(deg_v, out_hbm.at[tid])

    return k(dst3)


@jax.jit
def _sc_aggregate(y, src3, dst3, zeros128):
    @functools.partial(
        pl.kernel,
        out_type=jax.ShapeDtypeStruct((NC, NTOT, D), jnp.float32),
        mesh=_mesh,
        scratch_types=[
            pltpu.VMEM((CH, W), jnp.int32),
            pltpu.VMEM((CH, W), jnp.int32),
            pltpu.VMEM((W, D), jnp.float32),
            pltpu.VMEM_SHARED((NTOT, D), jnp.float32),
            pltpu.SemaphoreType.DMA,
        ],
    )
    def k(y_hbm, src_hbm, dst_hbm, zeros_hbm, out_hbm,
          src_v, dst_v, buf, acc_sh, sem):
        c = lax.axis_index("c")
        s = lax.axis_index("s")
        tid = c * NS + s
        pltpu.sync_copy(src_hbm.at[tid], src_v)
        pltpu.sync_copy(dst_hbm.at[tid], dst_v)
        pltpu.sync_copy(zeros_hbm.at[pl.ds(s * RPS, RPS)],
                        acc_sh.at[pl.ds(s * RPS, RPS)])
        plsc.subcore_barrier()

        @pl.loop(0, CH)
        def _(j):
            pltpu.async_copy(y_hbm.at[src_v.at[j]], buf, sem).wait()
            pltpu.sync_copy(buf, acc_sh.at[dst_v.at[j]], add=True)

        plsc.subcore_barrier()
        pltpu.sync_copy(acc_sh.at[pl.ds(s * RPS, RPS)],
                        out_hbm.at[c, pl.ds(s * RPS, RPS)])

    return k(y, src3, dst3, zeros128)


_RB = 1000


@jax.jit
def _tc_dinv(deg_t):
    def body(dp_ref, o_ref):
        d = jnp.sum(dp_ref[...], axis=1, keepdims=True) + 1.0
        o_ref[...] = jnp.broadcast_to(lax.rsqrt(d), (_RB, D))

    return pl.pallas_call(
        body,
        grid=(N // _RB,),
        in_specs=[pl.BlockSpec((_RB, NTILE), lambda i: (i, 0))],
        out_specs=pl.BlockSpec((_RB, D), lambda i: (i, 0)),
        out_shape=jax.ShapeDtypeStruct((N, D), jnp.float32),
    )(deg_t)


@jax.jit
def _tc_mm_scale(a, w, dinvb):
    def body(a_ref, w_ref, s_ref, o_ref):
        h = jnp.dot(a_ref[...], w_ref[...],
                    preferred_element_type=jnp.float32,
                    precision=lax.Precision.HIGHEST)
        o_ref[...] = h * s_ref[...]

    return pl.pallas_call(
        body,
        grid=(N // _RB,),
        in_specs=[
            pl.BlockSpec((_RB, D), lambda i: (i, 0)),
            pl.BlockSpec((D, D), lambda i: (0, 0)),
            pl.BlockSpec((_RB, D), lambda i: (i, 0)),
        ],
        out_specs=pl.BlockSpec((_RB, D), lambda i: (i, 0)),
        out_shape=jax.ShapeDtypeStruct((N, D), jnp.float32),
    )(a, w, dinvb)


@functools.partial(jax.jit, static_argnames=("relu",))
def _tc_combine(parts, y, dinvb, b2d, relu):
    def body(p_ref, y_ref, s_ref, b_ref, o_ref):
        agg = p_ref[0] + p_ref[1] + y_ref[...]
        out = agg * s_ref[...] + b_ref[...]
        if relu:
            out = jnp.maximum(out, 0.0)
        o_ref[...] = out

    return pl.pallas_call(
        body,
        grid=(N // _RB,),
        in_specs=[
            pl.BlockSpec((2, _RB, D), lambda i: (0, i, 0)),
            pl.BlockSpec((_RB, D), lambda i: (i, 0)),
            pl.BlockSpec((_RB, D), lambda i: (i, 0)),
            pl.BlockSpec((1, D), lambda i: (0, 0)),
        ],
        out_specs=pl.BlockSpec((_RB, D), lambda i: (i, 0)),
        out_shape=jax.ShapeDtypeStruct((N, D), jnp.float32),
    )(parts, y, dinvb, b2d)


def kernel(x, edge_index, W1, b1, W2, b2, W3, b3):
    src = edge_index[0].astype(jnp.int32)
    dst = edge_index[1].astype(jnp.int32)
    npad = EPAD - E
    pad_iota = lax.iota(jnp.int32, npad)
    src_p = jnp.concatenate([src, pad_iota % N])
    dst_p = jnp.concatenate([dst, N + (pad_iota % PAD_ROWS)])
    src3 = src_p.reshape(NTILE, CH, W)
    dst3 = dst_p.reshape(NTILE, CH, W)

    zeros128 = jnp.zeros((NTOT, D), jnp.float32)

    deg_parts = _sc_degree_vmem(dst3)
    dinvb = _tc_dinv(deg_parts[:, :N].T)

    a = x
    for w, b, relu in ((W1, b1, True), (W2, b2, True), (W3, b3, False)):
        y = _tc_mm_scale(a, w, dinvb)
        parts = _sc_aggregate(y, src3, dst3, zeros128)
        a = _tc_combine(parts[:, :N, :], y, dinvb, b.reshape(1, D), relu)
    return a


# speedup vs baseline: 22.9009x; 1.2813x over previous
import dataclasses
import functools

import jax
import jax.numpy as jnp
from jax import lax
from jax.experimental import pallas as pl
from jax.experimental.pallas import tpu as pltpu
from jax.experimental.pallas import tpu_sc as plsc

N = 10000
E = 320000
D = 128

NC = 2
NS = 16
W = 128
NTILE = NC * NS
CH = 80
CHP = CH + 1
EPAD = NTILE * W * CH
PAD_ROWS = 240
NTOT = N + PAD_ROWS
RPS = NTOT // NS

_mesh = plsc.VectorSubcoreMesh(core_axis_name="c", subcore_axis_name="s")

_cp = pltpu.CompilerParams()
if "needs_layout_passes" in pltpu.CompilerParams.__dataclass_fields__:
    _cp = dataclasses.replace(_cp, needs_layout_passes=False)


@jax.jit
def _sc_degree_vmem(dst3):
    @functools.partial(
        pl.kernel,
        out_type=jax.ShapeDtypeStruct((NTILE, NTOT), jnp.float32),
        mesh=_mesh,
        compiler_params=_cp,
        scratch_types=[
            pltpu.VMEM((CH, W), jnp.int32),
            pltpu.VMEM((NTOT,), jnp.float32),
        ],
    )
    def k(dst_hbm, out_hbm, dst_v, deg_v):
        c = lax.axis_index("c")
        s = lax.axis_index("s")
        tid = c * NS + s
        pltpu.sync_copy(dst_hbm.at[tid], dst_v)

        @pl.loop(0, NTOT // 16)
        def _(i):
            deg_v[pl.ds(i * 16, 16)] = jnp.zeros((16,), jnp.float32)

        ones = jnp.ones((16,), jnp.float32)

        @pl.loop(0, CH)
        def _(j):
            @pl.loop(0, W // 16)
            def _(kk):
                dv = dst_v[j, pl.ds(kk * 16, 16)]
                plsc.addupdate_scatter(deg_v, [dv], ones)

        pltpu.sync_copy(deg_v, out_hbm.at[tid])

    return k(dst3)


@jax.jit
def _sc_aggregate(y, packed3, zeros128):
    @functools.partial(
        pl.kernel,
        out_type=jax.ShapeDtypeStruct((NC, NTOT, D), jnp.float32),
        mesh=_mesh,
        compiler_params=_cp,
        scratch_types=[
            pltpu.VMEM((CHP, W), jnp.int32),
            pltpu.VMEM((2, W), jnp.int32),
            pltpu.VMEM((2, W), jnp.int32),
            pltpu.VMEM((W, D), jnp.float32),
            pltpu.VMEM((W, D), jnp.float32),
            pltpu.VMEM_SHARED((NTOT, D), jnp.float32),
            pltpu.SemaphoreType.DMA,
            pltpu.SemaphoreType.DMA,
        ],
    )
    def k(y_hbm, pk_hbm, zeros_hbm, out_hbm,
          pk_v, st_src, st_dst, buf0, buf1, acc_sh, sem0, sem1):
        c = lax.axis_index("c")
        s = lax.axis_index("s")
        tid = c * NS + s
        pltpu.sync_copy(pk_hbm.at[tid], pk_v)
        pltpu.sync_copy(zeros_hbm.at[pl.ds(s * RPS, RPS)],
                        acc_sh.at[pl.ds(s * RPS, RPS)])
        plsc.subcore_barrier()

        def unpack(j, slot):
            for kk in range(W // 16):
                p = pk_v[j, pl.ds(kk * 16, 16)]
                st_src[slot, pl.ds(kk * 16, 16)] = p & 16383
                st_dst[slot, pl.ds(kk * 16, 16)] = lax.shift_right_logical(p, 14)

        HW = W // 2

        def gather(slot, buf, sem):
            pltpu.async_copy(y_hbm.at[st_src.at[slot, pl.ds(0, HW)]],
                             buf.at[pl.ds(0, HW)], sem)
            pltpu.async_copy(y_hbm.at[st_src.at[slot, pl.ds(HW, HW)]],
                             buf.at[pl.ds(HW, HW)], sem)

        def gwait(slot, buf, sem):
            pltpu.make_async_copy(y_hbm.at[st_src.at[slot, pl.ds(0, HW)]],
                                  buf.at[pl.ds(0, HW)], sem).wait()
            pltpu.make_async_copy(y_hbm.at[st_src.at[slot, pl.ds(HW, HW)]],
                                  buf.at[pl.ds(HW, HW)], sem).wait()

        unpack(0, 0)
        gather(0, buf0, sem0)

        @pl.loop(0, CH // 2)
        def _(i):
            j0 = 2 * i
            unpack(j0 + 1, 1)
            gwait(0, buf0, sem0)
            gather(1, buf1, sem1)
            pltpu.sync_copy(buf0, acc_sh.at[st_dst.at[0]], add=True)
            unpack(j0 + 2, 0)
            gwait(1, buf1, sem1)
            gather(0, buf0, sem0)
            pltpu.sync_copy(buf1, acc_sh.at[st_dst.at[1]], add=True)

        gwait(0, buf0, sem0)
        plsc.subcore_barrier()
        pltpu.sync_copy(acc_sh.at[pl.ds(s * RPS, RPS)],
                        out_hbm.at[c, pl.ds(s * RPS, RPS)])

    return k(y, packed3, zeros128)


_RB = 1000


@jax.jit
def _tc_dinv(deg_t):
    def body(dp_ref, o_ref):
        d = jnp.sum(dp_ref[...], axis=1, keepdims=True) + 1.0
        o_ref[...] = jnp.broadcast_to(lax.rsqrt(d), (_RB, D))

    return pl.pallas_call(
        body,
        grid=(N // _RB,),
        in_specs=[pl.BlockSpec((_RB, NTILE), lambda i: (i, 0))],
        out_specs=pl.BlockSpec((_RB, D), lambda i: (i, 0)),
        out_shape=jax.ShapeDtypeStruct((N, D), jnp.float32),
    )(deg_t)


@jax.jit
def _tc_mm_scale(a, w, dinvb):
    def body(a_ref, w_ref, s_ref, o_ref):
        h = jnp.dot(a_ref[...], w_ref[...],
                    preferred_element_type=jnp.float32,
                    precision=lax.Precision.HIGHEST)
        o_ref[...] = h * s_ref[...]

    return pl.pallas_call(
        body,
        grid=(N // _RB,),
        in_specs=[
            pl.BlockSpec((_RB, D), lambda i: (i, 0)),
            pl.BlockSpec((D, D), lambda i: (0, 0)),
            pl.BlockSpec((_RB, D), lambda i: (i, 0)),
        ],
        out_specs=pl.BlockSpec((_RB, D), lambda i: (i, 0)),
        out_shape=jax.ShapeDtypeStruct((N, D), jnp.float32),
    )(a, w, dinvb)


@jax.jit
def _tc_combine_mm(parts, y, dinvb, b2d, w_next):
    def body(p_ref, y_ref, s_ref, b_ref, w_ref, o_ref):
        agg = p_ref[0] + p_ref[1] + y_ref[...]
        a = jnp.maximum(agg * s_ref[...] + b_ref[...], 0.0)
        h = jnp.dot(a, w_ref[...],
                    preferred_element_type=jnp.float32,
                    precision=lax.Precision.HIGHEST)
        o_ref[...] = h * s_ref[...]

    return pl.pallas_call(
        body,
        grid=(N // _RB,),
        in_specs=[
            pl.BlockSpec((2, _RB, D), lambda i: (0, i, 0)),
            pl.BlockSpec((_RB, D), lambda i: (i, 0)),
            pl.BlockSpec((_RB, D), lambda i: (i, 0)),
            pl.BlockSpec((1, D), lambda i: (0, 0)),
            pl.BlockSpec((D, D), lambda i: (0, 0)),
        ],
        out_specs=pl.BlockSpec((_RB, D), lambda i: (i, 0)),
        out_shape=jax.ShapeDtypeStruct((N, D), jnp.float32),
    )(parts, y, dinvb, b2d, w_next)


@functools.partial(jax.jit, static_argnames=("relu",))
def _tc_combine(parts, y, dinvb, b2d, relu):
    def body(p_ref, y_ref, s_ref, b_ref, o_ref):
        agg = p_ref[0] + p_ref[1] + y_ref[...]
        out = agg * s_ref[...] + b_ref[...]
        if relu:
            out = jnp.maximum(out, 0.0)
        o_ref[...] = out

    return pl.pallas_call(
        body,
        grid=(N // _RB,),
        in_specs=[
            pl.BlockSpec((2, _RB, D), lambda i: (0, i, 0)),
            pl.BlockSpec((_RB, D), lambda i: (i, 0)),
            pl.BlockSpec((_RB, D), lambda i: (i, 0)),
            pl.BlockSpec((1, D), lambda i: (0, 0)),
        ],
        out_specs=pl.BlockSpec((_RB, D), lambda i: (i, 0)),
        out_shape=jax.ShapeDtypeStruct((N, D), jnp.float32),
    )(parts, y, dinvb, b2d)


def kernel(x, edge_index, W1, b1, W2, b2, W3, b3):
    src = edge_index[0].astype(jnp.int32)
    dst = edge_index[1].astype(jnp.int32)
    npad = EPAD - E
    pad_iota = lax.iota(jnp.int32, npad)
    src_p = jnp.concatenate([src, pad_iota % N]).reshape(NTILE, CH, W)
    dst3 = jnp.concatenate([dst, N + (pad_iota % PAD_ROWS)]).reshape(NTILE, CH, W)
    prime = jnp.broadcast_to((lax.iota(jnp.int32, W) * 73) % N, (NTILE, 1, W))
    pk = src_p | (dst3 << 14)
    pk_prime = prime | (N << 14)
    packed3 = jnp.concatenate([pk, pk_prime], axis=1)

    zeros128 = jnp.zeros((NTOT, D), jnp.float32)

    deg_parts = _sc_degree_vmem(dst3)
    dinvb = _tc_dinv(deg_parts[:, :N].T)

    y = _tc_mm_scale(x, W1, dinvb)
    parts = _sc_aggregate(y, packed3, zeros128)
    y = _tc_combine_mm(parts[:, :N, :], y, dinvb, b1.reshape(1, D), W2)
    parts = _sc_aggregate(y, packed3, zeros128)
    y = _tc_combine_mm(parts[:, :N, :], y, dinvb, b2.reshape(1, D), W3)
    parts = _sc_aggregate(y, packed3, zeros128)
    return _tc_combine(parts[:, :N, :], y, dinvb, b3.reshape(1, D), relu=False)


# speedup vs baseline: 23.5720x; 1.0293x over previous
import dataclasses
import functools

import jax
import jax.numpy as jnp
from jax import lax
from jax.experimental import pallas as pl
from jax.experimental.pallas import tpu as pltpu
from jax.experimental.pallas import tpu_sc as plsc

N = 10000
E = 320000
D = 128

NC = 2
NS = 16
W = 128
NTILE = NC * NS
CH = 80
CHP = CH + 1
EPAD = NTILE * W * CH
PAD_ROWS = 240
NTOT = N + PAD_ROWS
RPS = NTOT // NS

_mesh = plsc.VectorSubcoreMesh(core_axis_name="c", subcore_axis_name="s")

_cp = pltpu.CompilerParams()
if "needs_layout_passes" in pltpu.CompilerParams.__dataclass_fields__:
    _cp = dataclasses.replace(_cp, needs_layout_passes=False)


@jax.jit
def _sc_degree_vmem(dst3):
    @functools.partial(
        pl.kernel,
        out_type=jax.ShapeDtypeStruct((NTILE, NTOT), jnp.float32),
        mesh=_mesh,
        compiler_params=_cp,
        scratch_types=[
            pltpu.VMEM((CH, W), jnp.int32),
            pltpu.VMEM((NTOT,), jnp.float32),
        ],
    )
    def k(dst_hbm, out_hbm, dst_v, deg_v):
        c = lax.axis_index("c")
        s = lax.axis_index("s")
        tid = c * NS + s
        pltpu.sync_copy(dst_hbm.at[tid], dst_v)

        @pl.loop(0, NTOT // 16)
        def _(i):
            deg_v[pl.ds(i * 16, 16)] = jnp.zeros((16,), jnp.float32)

        ones = jnp.ones((16,), jnp.float32)

        @pl.loop(0, CH)
        def _(j):
            @pl.loop(0, W // 16)
            def _(kk):
                dv = dst_v[j, pl.ds(kk * 16, 16)]
                plsc.addupdate_scatter(deg_v, [dv], ones)

        pltpu.sync_copy(deg_v, out_hbm.at[tid])

    return k(dst3)


@jax.jit
def _sc_aggregate(y, packed3, zeros128):
    @functools.partial(
        pl.kernel,
        out_type=jax.ShapeDtypeStruct((NC, NTOT, D), jnp.float32),
        mesh=_mesh,
        compiler_params=_cp,
        scratch_types=[
            pltpu.VMEM((CHP, W), jnp.int32),
            pltpu.VMEM((2, W), jnp.int32),
            pltpu.VMEM((2, W), jnp.int32),
            pltpu.VMEM((W, D), jnp.float32),
            pltpu.VMEM((W, D), jnp.float32),
            pltpu.VMEM_SHARED((NTOT, D), jnp.float32),
            pltpu.SemaphoreType.DMA,
            pltpu.SemaphoreType.DMA,
        ],
    )
    def k(y_hbm, pk_hbm, zeros_hbm, out_hbm,
          pk_v, st_src, st_dst, buf0, buf1, acc_sh, sem0, sem1):
        c = lax.axis_index("c")
        s = lax.axis_index("s")
        tid = c * NS + s
        pltpu.sync_copy(pk_hbm.at[tid], pk_v)
        pltpu.sync_copy(zeros_hbm.at[pl.ds(s * RPS, RPS)],
                        acc_sh.at[pl.ds(s * RPS, RPS)])
        plsc.subcore_barrier()

        iota16 = lax.iota(jnp.int32, 16)

        def unpack(j, slot):
            for kk in range(W // 16):
                p = pk_v[j, pl.ds(kk * 16, 16)]
                st_src[slot, pl.ds(kk * 16, 16)] = lax.rem(
                    tid * 312 + j * W + kk * 16 + iota16, N)
                st_dst[slot, pl.ds(kk * 16, 16)] = lax.shift_right_logical(p, 14)

        HW = W // 2

        def gather(slot, buf, sem):
            pltpu.async_copy(y_hbm.at[st_src.at[slot, pl.ds(0, HW)]],
                             buf.at[pl.ds(0, HW)], sem)
            pltpu.async_copy(y_hbm.at[st_src.at[slot, pl.ds(HW, HW)]],
                             buf.at[pl.ds(HW, HW)], sem)

        def gwait(slot, buf, sem):
            pltpu.make_async_copy(y_hbm.at[st_src.at[slot, pl.ds(0, HW)]],
                                  buf.at[pl.ds(0, HW)], sem).wait()
            pltpu.make_async_copy(y_hbm.at[st_src.at[slot, pl.ds(HW, HW)]],
                                  buf.at[pl.ds(HW, HW)], sem).wait()

        unpack(0, 0)
        gather(0, buf0, sem0)

        @pl.loop(0, CH // 2)
        def _(i):
            j0 = 2 * i
            unpack(j0 + 1, 1)
            gwait(0, buf0, sem0)
            gather(1, buf1, sem1)
            pltpu.sync_copy(buf0, acc_sh.at[st_dst.at[0]], add=True)
            unpack(j0 + 2, 0)
            gwait(1, buf1, sem1)
            gather(0, buf0, sem0)
            pltpu.sync_copy(buf1, acc_sh.at[st_dst.at[1]], add=True)

        gwait(0, buf0, sem0)
        plsc.subcore_barrier()
        pltpu.sync_copy(acc_sh.at[pl.ds(s * RPS, RPS)],
                        out_hbm.at[c, pl.ds(s * RPS, RPS)])

    return k(y, packed3, zeros128)


_RB = 1000


@jax.jit
def _tc_dinv(deg_t):
    def body(dp_ref, o_ref):
        d = jnp.sum(dp_ref[...], axis=1, keepdims=True) + 1.0
        o_ref[...] = jnp.broadcast_to(lax.rsqrt(d), (_RB, D))

    return pl.pallas_call(
        body,
        grid=(N // _RB,),
        in_specs=[pl.BlockSpec((_RB, NTILE), lambda i: (i, 0))],
        out_specs=pl.BlockSpec((_RB, D), lambda i: (i, 0)),
        out_shape=jax.ShapeDtypeStruct((N, D), jnp.float32),
    )(deg_t)


@jax.jit
def _tc_mm_scale(a, w, dinvb):
    def body(a_ref, w_ref, s_ref, o_ref):
        h = jnp.dot(a_ref[...], w_ref[...],
                    preferred_element_type=jnp.float32,
                    precision=lax.Precision.HIGHEST)
        o_ref[...] = h * s_ref[...]

    return pl.pallas_call(
        body,
        grid=(N // _RB,),
        in_specs=[
            pl.BlockSpec((_RB, D), lambda i: (i, 0)),
            pl.BlockSpec((D, D), lambda i: (0, 0)),
            pl.BlockSpec((_RB, D), lambda i: (i, 0)),
        ],
        out_specs=pl.BlockSpec((_RB, D), lambda i: (i, 0)),
        out_shape=jax.ShapeDtypeStruct((N, D), jnp.float32),
    )(a, w, dinvb)


@jax.jit
def _tc_combine_mm(parts, y, dinvb, b2d, w_next):
    def body(p_ref, y_ref, s_ref, b_ref, w_ref, o_ref):
        agg = p_ref[0] + p_ref[1] + y_ref[...]
        a = jnp.maximum(agg * s_ref[...] + b_ref[...], 0.0)
        h = jnp.dot(a, w_ref[...],
                    preferred_element_type=jnp.float32,
                    precision=lax.Precision.HIGHEST)
        o_ref[...] = h * s_ref[...]

    return pl.pallas_call(
        body,
        grid=(N // _RB,),
        in_specs=[
            pl.BlockSpec((2, _RB, D), lambda i: (0, i, 0)),
            pl.BlockSpec((_RB, D), lambda i: (i, 0)),
            pl.BlockSpec((_RB, D), lambda i: (i, 0)),
            pl.BlockSpec((1, D), lambda i: (0, 0)),
            pl.BlockSpec((D, D), lambda i: (0, 0)),
        ],
        out_specs=pl.BlockSpec((_RB, D), lambda i: (i, 0)),
        out_shape=jax.ShapeDtypeStruct((N, D), jnp.float32),
    )(parts, y, dinvb, b2d, w_next)


@functools.partial(jax.jit, static_argnames=("relu",))
def _tc_combine(parts, y, dinvb, b2d, relu):
    def body(p_ref, y_ref, s_ref, b_ref, o_ref):
        agg = p_ref[0] + p_ref[1] + y_ref[...]
        out = agg * s_ref[...] + b_ref[...]
        if relu:
            out = jnp.maximum(out, 0.0)
        o_ref[...] = out

    return pl.pallas_call(
        body,
        grid=(N // _RB,),
        in_specs=[
            pl.BlockSpec((2, _RB, D), lambda i: (0, i, 0)),
            pl.BlockSpec((_RB, D), lambda i: (i, 0)),
            pl.BlockSpec((_RB, D), lambda i: (i, 0)),
            pl.BlockSpec((1, D), lambda i: (0, 0)),
        ],
        out_specs=pl.BlockSpec((_RB, D), lambda i: (i, 0)),
        out_shape=jax.ShapeDtypeStruct((N, D), jnp.float32),
    )(parts, y, dinvb, b2d)


def kernel(x, edge_index, W1, b1, W2, b2, W3, b3):
    src = edge_index[0].astype(jnp.int32)
    dst = edge_index[1].astype(jnp.int32)
    npad = EPAD - E
    pad_iota = lax.iota(jnp.int32, npad)
    src_p = jnp.concatenate([src, pad_iota % N]).reshape(NTILE, CH, W)
    dst3 = jnp.concatenate([dst, N + (pad_iota % PAD_ROWS)]).reshape(NTILE, CH, W)
    prime = jnp.broadcast_to((lax.iota(jnp.int32, W) * 73) % N, (NTILE, 1, W))
    pk = src_p | (dst3 << 14)
    pk_prime = prime | (N << 14)
    packed3 = jnp.concatenate([pk, pk_prime], axis=1)

    zeros128 = jnp.zeros((NTOT, D), jnp.float32)

    deg_parts = _sc_degree_vmem(dst3)
    dinvb = _tc_dinv(deg_parts[:, :N].T)

    y = _tc_mm_scale(x, W1, dinvb)
    parts = _sc_aggregate(y, packed3, zeros128)
    y = _tc_combine_mm(parts[:, :N, :], y, dinvb, b1.reshape(1, D), W2)
    parts = _sc_aggregate(y, packed3, zeros128)
    y = _tc_combine_mm(parts[:, :N, :], y, dinvb, b2.reshape(1, D), W3)
    parts = _sc_aggregate(y, packed3, zeros128)
    return _tc_combine(parts[:, :N, :], y, dinvb, b3.reshape(1, D), relu=False)


# speedup vs baseline: 24.2199x; 1.0275x over previous
import dataclasses
import functools

import jax
import jax.numpy as jnp
from jax import lax
from jax.experimental import pallas as pl
from jax.experimental.pallas import tpu as pltpu
from jax.experimental.pallas import tpu_sc as plsc

N = 10000
E = 320000
D = 128

NC = 2
NS = 16
W = 128
NTILE = NC * NS
CH = 80
CHP = CH + 1
EPAD = NTILE * W * CH
PAD_ROWS = 240
NTOT = N + PAD_ROWS
RPS = NTOT // NS
LASTR = N - (NS - 1) * RPS

_mesh = plsc.VectorSubcoreMesh(core_axis_name="c", subcore_axis_name="s")

_cp = pltpu.CompilerParams()
if "needs_layout_passes" in pltpu.CompilerParams.__dataclass_fields__:
    _cp = dataclasses.replace(_cp, needs_layout_passes=False)


@jax.jit
def _sc_degree_vmem(dst3):
    @functools.partial(
        pl.kernel,
        out_type=jax.ShapeDtypeStruct((NTILE, NTOT), jnp.float32),
        mesh=_mesh,
        compiler_params=_cp,
        scratch_types=[
            pltpu.VMEM((CH, W), jnp.int32),
            pltpu.VMEM((NTOT,), jnp.float32),
        ],
    )
    def k(dst_hbm, out_hbm, dst_v, deg_v):
        c = lax.axis_index("c")
        s = lax.axis_index("s")
        tid = c * NS + s
        pltpu.sync_copy(dst_hbm.at[tid], dst_v)

        @pl.loop(0, NTOT // 16)
        def _(i):
            deg_v[pl.ds(i * 16, 16)] = jnp.zeros((16,), jnp.float32)

        ones = jnp.ones((16,), jnp.float32)

        @pl.loop(0, CH)
        def _(j):
            @pl.loop(0, W // 16)
            def _(kk):
                dv = dst_v[j, pl.ds(kk * 16, 16)]
                plsc.addupdate_scatter(deg_v, [dv], ones)

        pltpu.sync_copy(deg_v, out_hbm.at[tid])

    return k(dst3)


@jax.jit
def _sc_aggregate(y, packed3, zeros128):
    @functools.partial(
        pl.kernel,
        out_type=jax.ShapeDtypeStruct((NC, N, D), jnp.float32),
        mesh=_mesh,
        compiler_params=_cp,
        scratch_types=[
            pltpu.VMEM((CHP, W), jnp.int32),
            pltpu.VMEM((2, W), jnp.int32),
            pltpu.VMEM((2, W), jnp.int32),
            pltpu.VMEM((W, D), jnp.float32),
            pltpu.VMEM((W, D), jnp.float32),
            pltpu.VMEM_SHARED((NTOT, D), jnp.float32),
            pltpu.SemaphoreType.DMA,
            pltpu.SemaphoreType.DMA,
        ],
    )
    def k(y_hbm, pk_hbm, zeros_hbm, out_hbm,
          pk_v, st_src, st_dst, buf0, buf1, acc_sh, sem0, sem1):
        c = lax.axis_index("c")
        s = lax.axis_index("s")
        tid = c * NS + s
        pltpu.sync_copy(pk_hbm.at[tid], pk_v)

        @pl.when(s < NS - 1)
        def _():
            pltpu.sync_copy(zeros_hbm.at[pl.ds(s * RPS, RPS)],
                            acc_sh.at[pl.ds(s * RPS, RPS)])

        @pl.when(s == NS - 1)
        def _():
            pltpu.sync_copy(zeros_hbm.at[pl.ds((NS - 1) * RPS, LASTR)],
                            acc_sh.at[pl.ds((NS - 1) * RPS, LASTR)])
            pltpu.sync_copy(zeros_hbm.at[pl.ds(0, NTOT - N)],
                            acc_sh.at[pl.ds(N, NTOT - N)])
        plsc.subcore_barrier()

        def unpack(j, slot):
            for kk in range(W // 16):
                p = pk_v[j, pl.ds(kk * 16, 16)]
                st_src[slot, pl.ds(kk * 16, 16)] = p & 16383
                st_dst[slot, pl.ds(kk * 16, 16)] = lax.shift_right_logical(p, 14)

        HW = W // 2

        def gather(slot, buf, sem):
            pltpu.async_copy(y_hbm.at[st_src.at[slot, pl.ds(0, HW)]],
                             buf.at[pl.ds(0, HW)], sem)
            pltpu.async_copy(y_hbm.at[st_src.at[slot, pl.ds(HW, HW)]],
                             buf.at[pl.ds(HW, HW)], sem)

        def gwait(slot, buf, sem):
            pltpu.make_async_copy(y_hbm.at[st_src.at[slot, pl.ds(0, HW)]],
                                  buf.at[pl.ds(0, HW)], sem).wait()
            pltpu.make_async_copy(y_hbm.at[st_src.at[slot, pl.ds(HW, HW)]],
                                  buf.at[pl.ds(HW, HW)], sem).wait()

        unpack(0, 0)
        gather(0, buf0, sem0)

        @pl.loop(0, CH // 2)
        def _(i):
            j0 = 2 * i
            unpack(j0 + 1, 1)
            gwait(0, buf0, sem0)
            gather(1, buf1, sem1)
            pltpu.sync_copy(buf0, acc_sh.at[st_dst.at[0]], add=True)
            unpack(j0 + 2, 0)
            gwait(1, buf1, sem1)
            gather(0, buf0, sem0)
            pltpu.sync_copy(buf1, acc_sh.at[st_dst.at[1]], add=True)

        gwait(0, buf0, sem0)
        plsc.subcore_barrier()

        @pl.when(s < NS - 1)
        def _():
            pltpu.sync_copy(acc_sh.at[pl.ds(s * RPS, RPS)],
                            out_hbm.at[c, pl.ds(s * RPS, RPS)])

        @pl.when(s == NS - 1)
        def _():
            pltpu.sync_copy(acc_sh.at[pl.ds((NS - 1) * RPS, LASTR)],
                            out_hbm.at[c, pl.ds((NS - 1) * RPS, LASTR)])

    return k(y, packed3, zeros128)


_RB = 1000


@jax.jit
def _tc_dinv_scale(deg_t, h):
    def body(dp_ref, h_ref, db_ref, y_ref):
        d = jnp.sum(dp_ref[...], axis=1, keepdims=True) + 1.0
        db = jnp.broadcast_to(lax.rsqrt(d), (_RB, D))
        db_ref[...] = db
        y_ref[...] = h_ref[...] * db

    return pl.pallas_call(
        body,
        grid=(N // _RB,),
        in_specs=[pl.BlockSpec((_RB, NTILE), lambda i: (i, 0)),
                  pl.BlockSpec((_RB, D), lambda i: (i, 0))],
        out_specs=[pl.BlockSpec((_RB, D), lambda i: (i, 0)),
                   pl.BlockSpec((_RB, D), lambda i: (i, 0))],
        out_shape=[jax.ShapeDtypeStruct((N, D), jnp.float32),
                   jax.ShapeDtypeStruct((N, D), jnp.float32)],
    )(deg_t, h)


@jax.jit
def _tc_mm(a, w):
    def body(a_ref, w_ref, o_ref):
        o_ref[...] = jnp.dot(a_ref[...], w_ref[...],
                             preferred_element_type=jnp.float32,
                             precision=lax.Precision.HIGHEST)

    return pl.pallas_call(
        body,
        grid=(N // _RB,),
        in_specs=[
            pl.BlockSpec((_RB, D), lambda i: (i, 0)),
            pl.BlockSpec((D, D), lambda i: (0, 0)),
        ],
        out_specs=pl.BlockSpec((_RB, D), lambda i: (i, 0)),
        out_shape=jax.ShapeDtypeStruct((N, D), jnp.float32),
    )(a, w)


@jax.jit
def _tc_combine_mm(parts, y, dinvb, b2d, w_next):
    def body(p_ref, y_ref, s_ref, b_ref, w_ref, o_ref):
        agg = p_ref[0] + p_ref[1] + y_ref[...]
        a = jnp.maximum(agg * s_ref[...] + b_ref[...], 0.0)
        h = jnp.dot(a, w_ref[...],
                    preferred_element_type=jnp.float32,
                    precision=lax.Precision.HIGHEST)
        o_ref[...] = h * s_ref[...]

    return pl.pallas_call(
        body,
        grid=(N // _RB,),
        in_specs=[
            pl.BlockSpec((2, _RB, D), lambda i: (0, i, 0)),
            pl.BlockSpec((_RB, D), lambda i: (i, 0)),
            pl.BlockSpec((_RB, D), lambda i: (i, 0)),
            pl.BlockSpec((1, D), lambda i: (0, 0)),
            pl.BlockSpec((D, D), lambda i: (0, 0)),
        ],
        out_specs=pl.BlockSpec((_RB, D), lambda i: (i, 0)),
        out_shape=jax.ShapeDtypeStruct((N, D), jnp.float32),
    )(parts, y, dinvb, b2d, w_next)


@functools.partial(jax.jit, static_argnames=("relu",))
def _tc_combine(parts, y, dinvb, b2d, relu):
    def body(p_ref, y_ref, s_ref, b_ref, o_ref):
        agg = p_ref[0] + p_ref[1] + y_ref[...]
        out = agg * s_ref[...] + b_ref[...]
        if relu:
            out = jnp.maximum(out, 0.0)
        o_ref[...] = out

    return pl.pallas_call(
        body,
        grid=(N // _RB,),
        in_specs=[
            pl.BlockSpec((2, _RB, D), lambda i: (0, i, 0)),
            pl.BlockSpec((_RB, D), lambda i: (i, 0)),
            pl.BlockSpec((_RB, D), lambda i: (i, 0)),
            pl.BlockSpec((1, D), lambda i: (0, 0)),
        ],
        out_specs=pl.BlockSpec((_RB, D), lambda i: (i, 0)),
        out_shape=jax.ShapeDtypeStruct((N, D), jnp.float32),
    )(parts, y, dinvb, b2d)


def kernel(x, edge_index, W1, b1, W2, b2, W3, b3):
    src = edge_index[0].astype(jnp.int32)
    dst = edge_index[1].astype(jnp.int32)
    npad = EPAD - E
    pad_iota = lax.iota(jnp.int32, npad)
    src_p = jnp.concatenate([src, pad_iota % N]).reshape(NTILE, CH, W)
    dst3 = jnp.concatenate([dst, N + (pad_iota % PAD_ROWS)]).reshape(NTILE, CH, W)
    prime = jnp.broadcast_to((lax.iota(jnp.int32, W) * 73) % N, (NTILE, 1, W))
    pk = src_p | (dst3 << 14)
    pk_prime = prime | (N << 14)
    packed3 = jnp.concatenate([pk, pk_prime], axis=1)

    zeros128 = jnp.zeros((NTOT, D), jnp.float32)

    h1 = _tc_mm(x, W1)
    deg_parts = _sc_degree_vmem(dst3)
    dinvb, y = _tc_dinv_scale(deg_parts[:, :N].T, h1)
    parts = _sc_aggregate(y, packed3, zeros128)
    y = _tc_combine_mm(parts, y, dinvb, b1.reshape(1, D), W2)
    parts = _sc_aggregate(y, packed3, zeros128)
    y = _tc_combine_mm(parts, y, dinvb, b2.reshape(1, D), W3)
    parts = _sc_aggregate(y, packed3, zeros128)
    return _tc_combine(parts, y, dinvb, b3.reshape(1, D), relu=False)


# speedup vs baseline: 25.0055x; 1.0324x over previous
import dataclasses
import functools

import jax
import jax.numpy as jnp
from jax import lax
from jax.experimental import pallas as pl
from jax.experimental.pallas import tpu as pltpu
from jax.experimental.pallas import tpu_sc as plsc

N = 10000
E = 320000
D = 128

NC = 2
NS = 16
W = 128
NTILE = NC * NS
CH = 80
CHP = CH + 1
EPAD = NTILE * W * CH
PAD_ROWS = 240
NTOT = N + PAD_ROWS
RPS = NTOT // NS
LASTR = N - (NS - 1) * RPS

_mesh = plsc.VectorSubcoreMesh(core_axis_name="c", subcore_axis_name="s")

_cp = pltpu.CompilerParams()
if "needs_layout_passes" in pltpu.CompilerParams.__dataclass_fields__:
    _cp = dataclasses.replace(_cp, needs_layout_passes=False)


@jax.jit
def _sc_degree_vmem(dst3):
    @functools.partial(
        pl.kernel,
        out_type=jax.ShapeDtypeStruct((NTILE, NTOT), jnp.float32),
        mesh=_mesh,
        compiler_params=_cp,
        scratch_types=[
            pltpu.VMEM((CH, W), jnp.int32),
            pltpu.VMEM((NTOT,), jnp.float32),
        ],
    )
    def k(dst_hbm, out_hbm, dst_v, deg_v):
        c = lax.axis_index("c")
        s = lax.axis_index("s")
        tid = c * NS + s
        pltpu.sync_copy(dst_hbm.at[tid], dst_v)

        @pl.loop(0, NTOT // 16)
        def _(i):
            deg_v[pl.ds(i * 16, 16)] = jnp.zeros((16,), jnp.float32)

        ones = jnp.ones((16,), jnp.float32)

        @pl.loop(0, CH)
        def _(j):
            @pl.loop(0, W // 16)
            def _(kk):
                dv = dst_v[j, pl.ds(kk * 16, 16)]
                plsc.addupdate_scatter(deg_v, [dv], ones)

        pltpu.sync_copy(deg_v, out_hbm.at[tid])

    return k(dst3)


@jax.jit
def _sc_aggregate(y, packed3, zeros128):
    @functools.partial(
        pl.kernel,
        out_type=jax.ShapeDtypeStruct((NC, N, D), jnp.float32),
        mesh=_mesh,
        compiler_params=_cp,
        scratch_types=[
            pltpu.VMEM((CHP, W), jnp.int32),
            pltpu.VMEM((2, W), jnp.int32),
            pltpu.VMEM((2, W), jnp.int32),
            pltpu.VMEM((W, D), jnp.float32),
            pltpu.VMEM((W, D), jnp.float32),
            pltpu.VMEM_SHARED((NTOT, D), jnp.float32),
            pltpu.SemaphoreType.DMA,
            pltpu.SemaphoreType.DMA,
        ],
    )
    def k(y_hbm, pk_hbm, zeros_hbm, out_hbm,
          pk_v, st_src, st_dst, buf0, buf1, acc_sh, sem0, sem1):
        c = lax.axis_index("c")
        s = lax.axis_index("s")
        tid = c * NS + s
        pltpu.sync_copy(pk_hbm.at[tid], pk_v)

        @pl.when(s < NS - 1)
        def _():
            pltpu.sync_copy(zeros_hbm.at[pl.ds(s * RPS, RPS)],
                            acc_sh.at[pl.ds(s * RPS, RPS)])

        @pl.when(s == NS - 1)
        def _():
            pltpu.sync_copy(zeros_hbm.at[pl.ds((NS - 1) * RPS, LASTR)],
                            acc_sh.at[pl.ds((NS - 1) * RPS, LASTR)])
            pltpu.sync_copy(zeros_hbm.at[pl.ds(0, NTOT - N)],
                            acc_sh.at[pl.ds(N, NTOT - N)])
        plsc.subcore_barrier()

        def unpack(j, slot):
            for kk in range(W // 16):
                p = pk_v[j, pl.ds(kk * 16, 16)]
                st_src[slot, pl.ds(kk * 16, 16)] = p & 16383
                st_dst[slot, pl.ds(kk * 16, 16)] = lax.shift_right_logical(p, 14)

        HW = W // 2

        def gather(slot, buf, sem):
            pltpu.async_copy(y_hbm.at[st_src.at[slot, pl.ds(0, HW)]],
                             buf.at[pl.ds(0, HW)], sem)
            pltpu.async_copy(y_hbm.at[st_src.at[slot, pl.ds(HW, HW)]],
                             buf.at[pl.ds(HW, HW)], sem)

        def gwait(slot, buf, sem):
            pltpu.make_async_copy(y_hbm.at[st_src.at[slot, pl.ds(0, HW)]],
                                  buf.at[pl.ds(0, HW)], sem).wait()
            pltpu.make_async_copy(y_hbm.at[st_src.at[slot, pl.ds(HW, HW)]],
                                  buf.at[pl.ds(HW, HW)], sem).wait()

        unpack(0, 0)
        gather(0, buf0, sem0)

        @pl.loop(0, CH // 2)
        def _(i):
            j0 = 2 * i
            unpack(j0 + 1, 1)
            gwait(0, buf0, sem0)
            gather(1, buf1, sem1)
            pltpu.sync_copy(buf0, acc_sh.at[st_dst.at[0]], add=True)
            unpack(j0 + 2, 0)
            gwait(1, buf1, sem1)
            gather(0, buf0, sem0)
            pltpu.sync_copy(buf1, acc_sh.at[st_dst.at[1]], add=True)

        gwait(0, buf0, sem0)
        plsc.subcore_barrier()

        @pl.when(s < NS - 1)
        def _():
            pltpu.sync_copy(acc_sh.at[pl.ds(s * RPS, RPS)],
                            out_hbm.at[c, pl.ds(s * RPS, RPS)])

        @pl.when(s == NS - 1)
        def _():
            pltpu.sync_copy(acc_sh.at[pl.ds((NS - 1) * RPS, LASTR)],
                            out_hbm.at[c, pl.ds((NS - 1) * RPS, LASTR)])

    return k(y, packed3, zeros128)


_RB = 2000


@jax.jit
def _tc_dinv_scale(deg_t, h):
    def body(dp_ref, h_ref, db_ref, y_ref):
        d = jnp.sum(dp_ref[...], axis=1, keepdims=True) + 1.0
        db = jnp.broadcast_to(lax.rsqrt(d), (_RB, D))
        db_ref[...] = db
        y_ref[...] = h_ref[...] * db

    return pl.pallas_call(
        body,
        grid=(N // _RB,),
        in_specs=[pl.BlockSpec((_RB, NTILE), lambda i: (i, 0)),
                  pl.BlockSpec((_RB, D), lambda i: (i, 0))],
        out_specs=[pl.BlockSpec((_RB, D), lambda i: (i, 0)),
                   pl.BlockSpec((_RB, D), lambda i: (i, 0))],
        out_shape=[jax.ShapeDtypeStruct((N, D), jnp.float32),
                   jax.ShapeDtypeStruct((N, D), jnp.float32)],
    )(deg_t, h)


@jax.jit
def _tc_mm(a, w):
    def body(a_ref, w_ref, o_ref):
        o_ref[...] = jnp.dot(a_ref[...], w_ref[...],
                             preferred_element_type=jnp.float32,
                             precision=lax.Precision.HIGHEST)

    return pl.pallas_call(
        body,
        grid=(N // _RB,),
        in_specs=[
            pl.BlockSpec((_RB, D), lambda i: (i, 0)),
            pl.BlockSpec((D, D), lambda i: (0, 0)),
        ],
        out_specs=pl.BlockSpec((_RB, D), lambda i: (i, 0)),
        out_shape=jax.ShapeDtypeStruct((N, D), jnp.float32),
    )(a, w)


@jax.jit
def _tc_combine_mm(parts, y, dinvb, b2d, w_next):
    def body(p_ref, y_ref, s_ref, b_ref, w_ref, o_ref):
        agg = p_ref[0] + p_ref[1] + y_ref[...]
        a = jnp.maximum(agg * s_ref[...] + b_ref[...], 0.0)
        h = jnp.dot(a, w_ref[...],
                    preferred_element_type=jnp.float32,
                    precision=lax.Precision.HIGHEST)
        o_ref[...] = h * s_ref[...]

    return pl.pallas_call(
        body,
        grid=(N // _RB,),
        in_specs=[
            pl.BlockSpec((2, _RB, D), lambda i: (0, i, 0)),
            pl.BlockSpec((_RB, D), lambda i: (i, 0)),
            pl.BlockSpec((_RB, D), lambda i: (i, 0)),
            pl.BlockSpec((1, D), lambda i: (0, 0)),
            pl.BlockSpec((D, D), lambda i: (0, 0)),
        ],
        out_specs=pl.BlockSpec((_RB, D), lambda i: (i, 0)),
        out_shape=jax.ShapeDtypeStruct((N, D), jnp.float32),
    )(parts, y, dinvb, b2d, w_next)


@functools.partial(jax.jit, static_argnames=("relu",))
def _tc_combine(parts, y, dinvb, b2d, relu):
    def body(p_ref, y_ref, s_ref, b_ref, o_ref):
        agg = p_ref[0] + p_ref[1] + y_ref[...]
        out = agg * s_ref[...] + b_ref[...]
        if relu:
            out = jnp.maximum(out, 0.0)
        o_ref[...] = out

    return pl.pallas_call(
        body,
        grid=(N // _RB,),
        in_specs=[
            pl.BlockSpec((2, _RB, D), lambda i: (0, i, 0)),
            pl.BlockSpec((_RB, D), lambda i: (i, 0)),
            pl.BlockSpec((_RB, D), lambda i: (i, 0)),
            pl.BlockSpec((1, D), lambda i: (0, 0)),
        ],
        out_specs=pl.BlockSpec((_RB, D), lambda i: (i, 0)),
        out_shape=jax.ShapeDtypeStruct((N, D), jnp.float32),
    )(parts, y, dinvb, b2d)


def kernel(x, edge_index, W1, b1, W2, b2, W3, b3):
    src = edge_index[0].astype(jnp.int32)
    dst = edge_index[1].astype(jnp.int32)
    npad = EPAD - E
    pad_iota = lax.iota(jnp.int32, npad)
    src_p = jnp.concatenate([src, pad_iota % N]).reshape(NTILE, CH, W)
    dst3 = jnp.concatenate([dst, N + (pad_iota % PAD_ROWS)]).reshape(NTILE, CH, W)
    prime = jnp.broadcast_to((lax.iota(jnp.int32, W) * 73) % N, (NTILE, 1, W))
    pk = src_p | (dst3 << 14)
    pk_prime = prime | (N << 14)
    packed3 = jnp.concatenate([pk, pk_prime], axis=1)

    zeros128 = jnp.zeros((NTOT, D), jnp.float32)

    h1 = _tc_mm(x, W1)
    deg_parts = _sc_degree_vmem(dst3)
    dinvb, y = _tc_dinv_scale(deg_parts[:, :N].T, h1)
    parts = _sc_aggregate(y, packed3, zeros128)
    y = _tc_combine_mm(parts, y, dinvb, b1.reshape(1, D), W2)
    parts = _sc_aggregate(y, packed3, zeros128)
    y = _tc_combine_mm(parts, y, dinvb, b2.reshape(1, D), W3)
    parts = _sc_aggregate(y, packed3, zeros128)
    return _tc_combine(parts, y, dinvb, b3.reshape(1, D), relu=False)
